# Initial kernel scaffold; baseline (speedup 1.0000x reference)
import jax
import jax.numpy as jnp
from jax.experimental import pallas as pl


def kernel(x, embed):
    raise NotImplementedError("write your pallas kernel here")


# speedup vs baseline: 3.6597x; 3.6597x over previous
import functools

import jax
import jax.numpy as jnp
from jax import lax
from jax.experimental import pallas as pl
from jax.experimental.pallas import tpu as pltpu
from jax.experimental.pallas import tpu_sc as plsc

B, N, D = 64, 1024, 32
C = 1024
T = B * N
BT = 2048


def _argmax_body(x_ref, e_ref, idx_ref):
    eb = e_ref[...]
    norm = jnp.sqrt(jnp.sum(eb * eb, axis=1, keepdims=True))
    en = eb / jnp.maximum(norm, 1e-12)
    xb = x_ref[...]
    xnorm = jnp.sqrt(jnp.sum(xb * xb, axis=1, keepdims=True))
    xn = xb / jnp.maximum(xnorm, 1e-12)
    scores = lax.dot_general(
        xn, en, (((1,), (1,)), ((), ())),
        preferred_element_type=jnp.float32,
    )
    m = jnp.max(scores, axis=1, keepdims=True)
    iota = lax.broadcasted_iota(jnp.int32, scores.shape, 1)
    idx_ref[...] = jnp.min(jnp.where(scores == m, iota, C), axis=1)


_NC, _NS = 2, 16
_NW = _NC * _NS
_RPW = T // _NW


@functools.cache
def _sc_gather():
    @functools.partial(
        pl.kernel,
        mesh=plsc.VectorSubcoreMesh(core_axis_name="c", subcore_axis_name="s"),
        out_type=jax.ShapeDtypeStruct((T, D), jnp.float32),
        compiler_params=pltpu.CompilerParams(use_tc_tiling_on_sc=False),
        scratch_types=[
            pltpu.VMEM((_RPW,), jnp.int32),
            pltpu.VMEM((_RPW, D), jnp.float32),
            pltpu.SemaphoreType.DMA,
        ],
    )
    def gather(table_hbm, idx_hbm, out_hbm, idx_v, rows_v, sem):
        wid = lax.axis_index("s") * _NC + lax.axis_index("c")
        base = wid * _RPW
        pltpu.sync_copy(idx_hbm.at[pl.ds(base, _RPW)], idx_v)
        pltpu.async_copy(table_hbm.at[idx_v], rows_v, sem).wait()
        pltpu.sync_copy(rows_v, out_hbm.at[pl.ds(base, _RPW)])

    return gather


def kernel(x, embed):
    xf = x.reshape(T, D)
    table = embed.reshape(C, D)
    idx = pl.pallas_call(
        _argmax_body,
        grid=(T // BT,),
        in_specs=[
            pl.BlockSpec((BT, D), lambda i: (i, 0)),
            pl.BlockSpec((C, D), lambda i: (0, 0)),
        ],
        out_specs=pl.BlockSpec((BT,), lambda i: (i,)),
        out_shape=jax.ShapeDtypeStruct((T,), jnp.int32),
    )(xf, table)
    quantize = _sc_gather()(table, idx)
    return quantize.reshape(B, N, D), idx.reshape(B, N)


# speedup vs baseline: 5.4714x; 1.4951x over previous
import functools

import jax
import jax.numpy as jnp
from jax import lax
from jax.experimental import pallas as pl
from jax.experimental.pallas import tpu as pltpu
from jax.experimental.pallas import tpu_sc as plsc

B, N, D = 64, 1024, 32
C = 1024
T = B * N
BT = 2048


def _argmax_body(x_ref, e_ref, idx_ref, en_ref):
    @pl.when(pl.program_id(0) == 0)
    def _():
        eb = e_ref[...]
        norm = jnp.sqrt(jnp.sum(eb * eb, axis=1, keepdims=True))
        en_ref[...] = eb / jnp.maximum(norm, 1e-12)

    xb = x_ref[...]
    xnorm = jnp.sqrt(jnp.sum(xb * xb, axis=1, keepdims=True))
    xn = xb / jnp.maximum(xnorm, 1e-12)
    scores = lax.dot_general(
        en_ref[...], xn, (((1,), (1,)), ((), ())),
        preferred_element_type=jnp.float32,
    )
    m = jnp.max(scores, axis=0, keepdims=True)
    iota = lax.broadcasted_iota(jnp.int32, (C, 1), 0).astype(jnp.float32)
    idx_ref[...] = jnp.min(
        jnp.where(scores == m, iota, float(C)), axis=0
    ).astype(jnp.int32)


_NC, _NS = 2, 16
_NW = _NC * _NS
_RPW = T // _NW


@functools.cache
def _sc_gather():
    @functools.partial(
        pl.kernel,
        mesh=plsc.VectorSubcoreMesh(core_axis_name="c", subcore_axis_name="s"),
        out_type=jax.ShapeDtypeStruct((T, D), jnp.float32),
        compiler_params=pltpu.CompilerParams(use_tc_tiling_on_sc=False),
        scratch_types=[
            pltpu.VMEM((_RPW,), jnp.int32),
            pltpu.VMEM((_RPW, D), jnp.float32),
            pltpu.SemaphoreType.DMA,
        ],
    )
    def gather(table_hbm, idx_hbm, out_hbm, idx_v, rows_v, sem):
        wid = lax.axis_index("s") * _NC + lax.axis_index("c")
        base = wid * _RPW
        pltpu.sync_copy(idx_hbm.at[pl.ds(base, _RPW)], idx_v)
        pltpu.async_copy(table_hbm.at[idx_v], rows_v, sem).wait()
        pltpu.sync_copy(rows_v, out_hbm.at[pl.ds(base, _RPW)])

    return gather


def kernel(x, embed):
    xf = x.reshape(T, D)
    table = embed.reshape(C, D)
    idx = pl.pallas_call(
        _argmax_body,
        grid=(T // BT,),
        in_specs=[
            pl.BlockSpec((BT, D), lambda i: (i, 0)),
            pl.BlockSpec((C, D), lambda i: (0, 0)),
        ],
        out_specs=pl.BlockSpec((BT,), lambda i: (i,)),
        out_shape=jax.ShapeDtypeStruct((T,), jnp.int32),
        scratch_shapes=[pltpu.VMEM((C, D), jnp.float32)],
    )(xf, table)
    quantize = _sc_gather()(table, idx)
    return quantize.reshape(B, N, D), idx.reshape(B, N)


# speedup vs baseline: 5.9196x; 1.0819x over previous
import functools

import jax
import jax.numpy as jnp
from jax import lax
from jax.experimental import pallas as pl
from jax.experimental.pallas import tpu as pltpu
from jax.experimental.pallas import tpu_sc as plsc

B, N, D = 64, 1024, 32
C = 1024
T = B * N
BT = 2048


def _argmax_body(x_ref, e_ref, idx_ref, en_ref):
    @pl.when(pl.program_id(0) == 0)
    def _():
        eb = e_ref[...]
        norm = jnp.sqrt(jnp.sum(eb * eb, axis=1, keepdims=True))
        en_ref[...] = eb / jnp.maximum(norm, 1e-12)

    xr = x_ref[...]
    x2 = jnp.concatenate([xr[b] for b in range(xr.shape[0])], axis=1)
    xnorm = jnp.sqrt(jnp.sum(x2 * x2, axis=0, keepdims=True))
    xn = x2 / jnp.maximum(xnorm, 1e-12)
    scores = lax.dot_general(
        en_ref[...], xn, (((1,), (0,)), ((), ())),
        preferred_element_type=jnp.float32,
    )
    m = jnp.max(scores, axis=0, keepdims=True)
    iota = lax.broadcasted_iota(jnp.int32, (C, 1), 0).astype(jnp.float32)
    idx_ref[...] = jnp.min(
        jnp.where(scores == m, iota, float(C)), axis=0
    ).astype(jnp.int32)


_NC, _NS = 2, 16
_NW = _NC * _NS
_RPW = T // _NW


_BPW = B // _NW


@functools.cache
def _sc_gather():
    @functools.partial(
        pl.kernel,
        mesh=plsc.VectorSubcoreMesh(core_axis_name="c", subcore_axis_name="s"),
        out_type=jax.ShapeDtypeStruct((B, D // 8, N // 128, 8, 128), jnp.float32),
        compiler_params=pltpu.CompilerParams(
            use_tc_tiling_on_sc=False, needs_layout_passes=False
        ),
        scratch_types=[
            pltpu.VMEM((C * D,), jnp.float32),
            pltpu.VMEM((_RPW,), jnp.int32),
            pltpu.VMEM((_BPW, D // 8, N // 128, 8, 128), jnp.float32),
            pltpu.SemaphoreType.DMA,
        ],
    )
    def gather(table_hbm, idx_hbm, out_hbm, table_v, idx_v, out_v, sem):
        wid = lax.axis_index("s") * _NC + lax.axis_index("c")
        base = wid * _RPW
        pltpu.sync_copy(idx_hbm.at[pl.ds(base, _RPW)], idx_v)
        pltpu.sync_copy(table_hbm, table_v)

        for bl in range(_BPW):

            def body(i, _):
                idx16 = idx_v[pl.ds(bl * N + i * 16, 16)]
                a32 = idx16 * D
                for dt in range(D // 8):
                    for dr in range(8):
                        vals = plsc.load_gather(table_v, [a32 + (dt * 8 + dr)])
                        out_v[bl, dt, i // 8, dr, pl.ds((i % 8) * 16, 16)] = vals
                return _

            lax.fori_loop(0, N // 16, body, 0)
        pltpu.sync_copy(out_v, out_hbm.at[pl.ds(_BPW * wid, _BPW)])

    return gather


def kernel(x, embed):
    xt = jnp.transpose(x, (0, 2, 1))
    table = embed.reshape(C, D)
    bb = BT // N
    idx = pl.pallas_call(
        _argmax_body,
        grid=(T // BT,),
        in_specs=[
            pl.BlockSpec((bb, D, N), lambda i: (i, 0, 0)),
            pl.BlockSpec((C, D), lambda i: (0, 0)),
        ],
        out_specs=pl.BlockSpec((BT,), lambda i: (i,)),
        out_shape=jax.ShapeDtypeStruct((T,), jnp.int32),
        scratch_shapes=[pltpu.VMEM((C, D), jnp.float32)],
    )(xt, table)
    q5 = _sc_gather()(embed.reshape(C * D), idx)
    quantize = q5.transpose(0, 2, 4, 1, 3).reshape(B, N, D)
    return quantize, idx.reshape(B, N)


# speedup vs baseline: 8.8775x; 1.4997x over previous
import functools

import jax
import jax.numpy as jnp
from jax import lax
from jax.experimental import pallas as pl
from jax.experimental.pallas import tpu as pltpu
from jax.experimental.pallas import tpu_sc as plsc

B, N, D = 64, 1024, 32
C = 1024
T = B * N
BT = 2048


def _argmax_body(x_ref, e_ref, idx_ref, en_ref):
    @pl.when(pl.program_id(0) == 0)
    def _():
        eb = e_ref[...]
        norm = jnp.sqrt(jnp.sum(eb * eb, axis=1, keepdims=True))
        en_ref[...] = eb / jnp.maximum(norm, 1e-12)

    xr = x_ref[...]
    x2 = jnp.concatenate([xr[b] for b in range(xr.shape[0])], axis=1)
    xnorm = jnp.sqrt(jnp.sum(x2 * x2, axis=0, keepdims=True))
    xn = x2 / jnp.maximum(xnorm, 1e-12)
    scores = lax.dot_general(
        en_ref[...], xn, (((1,), (0,)), ((), ())),
        preferred_element_type=jnp.float32,
    )
    m = jnp.max(scores, axis=0, keepdims=True)
    iota = lax.broadcasted_iota(jnp.int32, (C, 1), 0).astype(jnp.float32)
    idx_ref[...] = jnp.min(
        jnp.where(scores == m, iota, float(C)), axis=0
    ).astype(jnp.int32)


_NC, _NS = 2, 16
_NW = _NC * _NS
_RPW = T // _NW


_BPW = B // _NW


@functools.cache
def _sc_gather():
    @functools.partial(
        pl.kernel,
        mesh=plsc.VectorSubcoreMesh(core_axis_name="c", subcore_axis_name="s"),
        out_type=jax.ShapeDtypeStruct((B, D // 8, N // 128, 8, 128), jnp.float32),
        compiler_params=pltpu.CompilerParams(
            use_tc_tiling_on_sc=False, needs_layout_passes=False
        ),
        scratch_types=[
            pltpu.VMEM((C * D,), jnp.float32),
            pltpu.VMEM((_RPW,), jnp.int32),
            pltpu.VMEM((_BPW, D // 8, N // 128, 8, 128), jnp.float32),
            pltpu.SemaphoreType.DMA,
        ],
    )
    def gather(table_hbm, idx_hbm, out_hbm, table_v, idx_v, out_v, sem):
        wid = lax.axis_index("s") * _NC + lax.axis_index("c")
        base = wid * _RPW
        pltpu.sync_copy(idx_hbm.at[pl.ds(base, _RPW)], idx_v)
        pltpu.sync_copy(table_hbm, table_v)

        for bl in range(_BPW):

            @functools.partial(plsc.parallel_loop, 0, N // 128, unroll=2)
            def _(nt):
                for q in range(8):
                    idx16 = idx_v[pl.ds(bl * N + nt * 128 + q * 16, 16)]
                    a32 = idx16 * D
                    for d in range(D):
                        vals = plsc.load_gather(table_v, [a32 + d])
                        out_v[bl, d // 8, nt, d % 8, pl.ds(q * 16, 16)] = vals
        pltpu.sync_copy(out_v, out_hbm.at[pl.ds(_BPW * wid, _BPW)])

    return gather


def kernel(x, embed):
    xt = jnp.transpose(x, (0, 2, 1))
    table = embed.reshape(C, D)
    bb = BT // N
    idx = pl.pallas_call(
        _argmax_body,
        grid=(T // BT,),
        in_specs=[
            pl.BlockSpec((bb, D, N), lambda i: (i, 0, 0)),
            pl.BlockSpec((C, D), lambda i: (0, 0)),
        ],
        out_specs=pl.BlockSpec((BT,), lambda i: (i,)),
        out_shape=jax.ShapeDtypeStruct((T,), jnp.int32),
        scratch_shapes=[pltpu.VMEM((C, D), jnp.float32)],
    )(xt, table)
    q5 = _sc_gather()(embed.reshape(C * D), idx)
    quantize = q5.transpose(0, 2, 4, 1, 3).reshape(B, N, D)
    return quantize, idx.reshape(B, N)
